# baseline (speedup 1.0000x reference)
import functools

import jax
import jax.numpy as jnp
from jax import lax
from jax.experimental import pallas as pl
from jax.experimental.pallas import tpu as pltpu
from jax.experimental.pallas import tpu_sc as plsc

B = 128
BEAM = 4
VOCAB = 100000
NEG = -3.0e38
INTBIG = 2 ** 30

CHUNK = 400
NCH = VOCAB // CHUNK
NSTR = 8
NLG = 8
VUNROLL = 2

LSE_BLK = VOCAB // 20


def _lse_part_kernel(x_ref, m_ref, s_ref):
    x = x_ref[0]
    mx = jnp.max(x, axis=0, keepdims=True)
    s = jnp.sum(jnp.exp(x - mx), axis=0, keepdims=True)
    m_ref[...] = mx[None, None]
    s_ref[...] = s[None, None]


def _merge_kernel(cv_ref, ct_ref, mp_ref, sp_ref, bs_ref,
                  os_ref, ot_ref, ob_ref):
    cv = cv_ref[...]
    ct = ct_ref[...]
    mp = mp_ref[...]
    sp = sp_ref[...]
    bs = bs_ref[...]

    grp = jax.lax.broadcasted_iota(jnp.int32, mp.shape, 1) // 20
    slotbeam = jax.lax.broadcasted_iota(jnp.int32, cv.shape, 1) // 32

    adj = jnp.zeros_like(cv)
    for m in range(BEAM):
        sel = grp == m
        mb = jnp.max(jnp.where(sel, mp, NEG), axis=1, keepdims=True)
        sb = jnp.sum(jnp.where(sel, sp * jnp.exp(mp - mb), 0.0),
                     axis=1, keepdims=True)
        lse = mb + jnp.log(sb)
        adj = adj + jnp.where(slotbeam == m, bs[:, m:m + 1] - lse, 0.0)

    y = cv + adj
    fl = slotbeam * VOCAB + ct
    ss, tt, bb = [], [], []
    for _ in range(4):
        v = jnp.max(y, axis=1, keepdims=True)
        flb = jnp.min(jnp.where(y == v, fl, INTBIG), axis=1, keepdims=True)
        ss.append(v)
        tt.append(flb % VOCAB)
        bb.append(flb // VOCAB)
        y = jnp.where(fl == flb, NEG, y)
    os_ref[...] = jnp.concatenate(ss, axis=1)
    ot_ref[...] = jnp.concatenate(tt, axis=1)
    ob_ref[...] = jnp.concatenate(bb, axis=1)


def _sc_topk_body(x_hbm, vals_hbm, idx_hbm, buf_v, iv_v, tv_v, stv_v, sti_v, sem):
    wid = lax.axis_index("s") * 2 + lax.axis_index("c")
    m = wid // NSTR
    j = wid - m * NSTR
    nk = (NCH - 1 - j) // NSTR + 1
    lane = lax.broadcasted_iota(jnp.int32, (16,), 0)
    negv = jnp.full((16,), NEG, jnp.float32)

    def chunk_src(k):
        return x_hbm.at[m, pl.ds(k * CHUNK, CHUNK)]

    pltpu.async_copy(chunk_src(j), buf_v.at[0], sem)

    for lg in range(NLG):
        for r in range(3):
            tv_v[lg, r] = negv

    def chunk_body(c, tq3):
        k = j + NSTR * c
        sel = lax.rem(c, 2)
        knext = jnp.minimum(k + NSTR, NCH - 1)
        pltpu.async_copy(chunk_src(knext), buf_v.at[1 - sel], sem)
        pltpu.make_async_copy(chunk_src(k), buf_v.at[sel], sem).wait()

        def v_body(i, tq3):
            v0 = i * VUNROLL
            tors = []
            for dv in range(VUNROLL):
                tms = [buf_v[sel, v0 + dv, pl.ds(lg * 16, 16)] > tq3[lg]
                       for lg in range(NLG)]
                t01 = tms[0] | tms[1]
                t23 = tms[2] | tms[3]
                t45 = tms[4] | tms[5]
                t67 = tms[6] | tms[7]
                tors.append((t01 | t23) | (t45 | t67))
            tor = tors[0]
            for dv in range(1, VUNROLL):
                tor = tor | tors[dv]
            cnt = plsc.all_reduce_population_count(tor)[0]

            def slow(tq3):
                out = list(tq3)
                for dv in range(VUNROLL):
                    vabs = k * CHUNK + v0 + dv
                    for lg in range(NLG):
                        vv = buf_v[sel, v0 + dv, pl.ds(lg * 16, 16)]
                        clg = vv > out[lg]
                        cl = plsc.all_reduce_population_count(clg)[0]

                        def ins(q3, lg=lg, vv=vv, vabs=vabs):
                            q = [tv_v[lg, 0], tv_v[lg, 1], tv_v[lg, 2], q3]
                            t = vv
                            ti = lane * 0 + vabs
                            for r in range(4):
                                iv = iv_v[lg, r]
                                cc = t > q[r]
                                nv = jnp.where(cc, t, q[r])
                                ni = jnp.where(cc, ti, iv)
                                t = jnp.where(cc, q[r], t)
                                ti = jnp.where(cc, iv, ti)
                                q[r] = nv
                                iv_v[lg, r] = ni
                            for r in range(3):
                                tv_v[lg, r] = q[r]
                            return q[3]

                        out[lg] = lax.cond(cl > 0, ins, lambda q3: q3,
                                           out[lg])
                return tuple(out)

            return lax.cond(cnt > 0, slow, lambda tq3: tq3, tq3)

        return lax.fori_loop(0, CHUNK // VUNROLL, v_body, tq3)

    tq30 = (negv,) * NLG
    tq3 = lax.fori_loop(0, nk, chunk_body, tq30)
    pltpu.make_async_copy(chunk_src(0), buf_v.at[0], sem).wait()

    for lg in range(NLG):
        tv_v[lg, 3] = tq3[lg]
    for r in range(4):
        for lg in range(NLG):
            stv_v[r, pl.ds(lg * 16, 16)] = tv_v[lg, r]
            sti_v[r, pl.ds(lg * 16, 16)] = iv_v[lg, r]
    pltpu.sync_copy(stv_v, vals_hbm.at[wid])
    pltpu.sync_copy(sti_v, idx_hbm.at[wid])


@jax.jit
def kernel(logits, beam_scores):
    b, beam, vocab = logits.shape
    xT = jnp.transpose(logits, (1, 2, 0))

    sc_topk = functools.partial(
        pl.kernel,
        mesh=plsc.VectorSubcoreMesh(core_axis_name="c", subcore_axis_name="s"),
        compiler_params=pltpu.CompilerParams(
            needs_layout_passes=False, use_tc_tiling_on_sc=True),
        out_type=[
            jax.ShapeDtypeStruct((32, 4, b), jnp.float32),
            jax.ShapeDtypeStruct((32, 4, b), jnp.int32),
        ],
        scratch_types=[
            pltpu.VMEM((2, CHUNK, b), jnp.float32),
            pltpu.VMEM((NLG, 4, 16), jnp.int32),
            pltpu.VMEM((NLG, 4, 16), jnp.float32),
            pltpu.VMEM((4, b), jnp.float32),
            pltpu.VMEM((4, b), jnp.int32),
            pltpu.SemaphoreType.DMA,
        ],
    )(_sc_topk_body)
    cvals, cidx = sc_topk(xT)

    mpart, spart = pl.pallas_call(
        _lse_part_kernel,
        grid=(beam, vocab // LSE_BLK),
        in_specs=[pl.BlockSpec((1, LSE_BLK, b), lambda i, j: (i, j, 0))],
        out_specs=[
            pl.BlockSpec((1, 1, 1, b), lambda i, j: (i, j, 0, 0)),
            pl.BlockSpec((1, 1, 1, b), lambda i, j: (i, j, 0, 0)),
        ],
        out_shape=[
            jax.ShapeDtypeStruct((beam, vocab // LSE_BLK, 1, b), jnp.float32),
            jax.ShapeDtypeStruct((beam, vocab // LSE_BLK, 1, b), jnp.float32),
        ],
    )(xT)

    cv = cvals.transpose(2, 0, 1).reshape(b, 128)
    ct = cidx.transpose(2, 0, 1).reshape(b, 128)
    mp = mpart.reshape(beam, vocab // LSE_BLK, b).transpose(2, 0, 1).reshape(
        b, beam * (vocab // LSE_BLK))
    sp = spart.reshape(beam, vocab // LSE_BLK, b).transpose(2, 0, 1).reshape(
        b, beam * (vocab // LSE_BLK))

    os_, ot, ob = pl.pallas_call(
        _merge_kernel,
        out_shape=[
            jax.ShapeDtypeStruct((b, 4), jnp.float32),
            jax.ShapeDtypeStruct((b, 4), jnp.int32),
            jax.ShapeDtypeStruct((b, 4), jnp.int32),
        ],
    )(cv, ct, mp, sp, beam_scores)

    return os_, ot, ob


# speedup vs baseline: 1.7563x; 1.7563x over previous
import functools

import jax
import jax.numpy as jnp
from jax import lax
from jax.experimental import pallas as pl
from jax.experimental.pallas import tpu as pltpu
from jax.experimental.pallas import tpu_sc as plsc

B = 128
BEAM = 4
VOCAB = 100000
NEG = -3.0e38
INTBIG = 2 ** 30

CHUNK = 400
NCH = VOCAB // CHUNK
NSTR = 8
NLG = 8
VUNROLL = 2

LSE_BLK = VOCAB // 20


def _lse_part_kernel(x_ref, m_ref, s_ref):
    x = x_ref[0]
    mx = jnp.max(x, axis=0, keepdims=True)
    s = jnp.sum(jnp.exp(x - mx), axis=0, keepdims=True)
    m_ref[...] = mx[None, None]
    s_ref[...] = s[None, None]


def _merge_kernel(cv_ref, ct_ref, mp_ref, sp_ref, bs_ref,
                  os_ref, ot_ref, ob_ref):
    cv = cv_ref[...]
    ct = ct_ref[...]
    mp = mp_ref[...]
    sp = sp_ref[...]
    bs = bs_ref[...]

    grp = jax.lax.broadcasted_iota(jnp.int32, mp.shape, 1) // 20
    slotbeam = jax.lax.broadcasted_iota(jnp.int32, cv.shape, 1) // 32

    adj = jnp.zeros_like(cv)
    for m in range(BEAM):
        sel = grp == m
        mb = jnp.max(jnp.where(sel, mp, NEG), axis=1, keepdims=True)
        sb = jnp.sum(jnp.where(sel, sp * jnp.exp(mp - mb), 0.0),
                     axis=1, keepdims=True)
        lse = mb + jnp.log(sb)
        adj = adj + jnp.where(slotbeam == m, bs[:, m:m + 1] - lse, 0.0)

    y = cv + adj
    fl = slotbeam * VOCAB + ct
    ss, tt, bb = [], [], []
    for _ in range(4):
        v = jnp.max(y, axis=1, keepdims=True)
        flb = jnp.min(jnp.where(y == v, fl, INTBIG), axis=1, keepdims=True)
        ss.append(v)
        tt.append(flb % VOCAB)
        bb.append(flb // VOCAB)
        y = jnp.where(fl == flb, NEG, y)
    os_ref[...] = jnp.concatenate(ss, axis=1)
    ot_ref[...] = jnp.concatenate(tt, axis=1)
    ob_ref[...] = jnp.concatenate(bb, axis=1)


def _sc_topk_body(x_hbm, vals_hbm, idx_hbm, buf_v, iv_v, stv_v, sti_v, sem):
    wid = lax.axis_index("s") * 2 + lax.axis_index("c")
    m = wid // NSTR
    j = wid - m * NSTR
    nk = (NCH - 1 - j) // NSTR + 1
    lane = lax.broadcasted_iota(jnp.int32, (16,), 0)
    negv = jnp.full((16,), NEG, jnp.float32)

    def chunk_src(k):
        return x_hbm.at[m, pl.ds(k * CHUNK, CHUNK)]

    pltpu.async_copy(chunk_src(j), buf_v.at[0], sem)

    def chunk_body(c, carry):
        mm, tq = carry
        k = j + NSTR * c
        sel = lax.rem(c, 2)
        knext = jnp.minimum(k + NSTR, NCH - 1)
        pltpu.async_copy(chunk_src(knext), buf_v.at[1 - sel], sem)
        pltpu.make_async_copy(chunk_src(k), buf_v.at[sel], sem).wait()

        def v_body(v, carry):
            mm, tq = carry
            vabs = k * CHUNK + v
            vs = [buf_v[sel, v, pl.ds(lg * 16, 16)] for lg in range(NLG)]
            mm = tuple(jnp.maximum(mm[lg], vs[lg]) for lg in range(NLG))
            tms = [vs[lg] > tq[lg * 4 + 3] for lg in range(NLG)]
            t01 = tms[0] | tms[1]
            t23 = tms[2] | tms[3]
            t45 = tms[4] | tms[5]
            t67 = tms[6] | tms[7]
            tor = (t01 | t23) | (t45 | t67)
            cnt = plsc.all_reduce_population_count(tor)[0]

            def slow(tq):
                out = list(tq)
                for lg in range(NLG):
                    clg = vs[lg] > tq[lg * 4 + 3]
                    cl = plsc.all_reduce_population_count(clg)[0]

                    def ins(q4, lg=lg):
                        q = list(q4)
                        t = vs[lg]
                        ti = lane * 0 + vabs
                        for r in range(4):
                            iv = iv_v[lg, r]
                            cc = t > q[r]
                            nv = jnp.where(cc, t, q[r])
                            ni = jnp.where(cc, ti, iv)
                            t = jnp.where(cc, q[r], t)
                            ti = jnp.where(cc, iv, ti)
                            q[r] = nv
                            iv_v[lg, r] = ni
                        return tuple(q)

                    q4 = lax.cond(cl > 0, ins, lambda q4: q4,
                                  tuple(out[lg * 4:lg * 4 + 4]))
                    out[lg * 4:lg * 4 + 4] = list(q4)
                return tuple(out)

            tq = lax.cond(cnt > 0, slow, lambda tq: tq, tq)
            return (mm, tq)

        return lax.fori_loop(0, CHUNK, v_body, (mm, tq))

    mm0 = (negv,) * NLG
    tq0 = (negv,) * (NLG * 4)
    mm, tq = lax.fori_loop(0, nk, chunk_body, (mm0, tq0))
    pltpu.make_async_copy(chunk_src(0), buf_v.at[0], sem).wait()

    for r in range(4):
        for lg in range(NLG):
            stv_v[r, pl.ds(lg * 16, 16)] = tq[lg * 4 + r]
            sti_v[r, pl.ds(lg * 16, 16)] = iv_v[lg, r]
    pltpu.sync_copy(stv_v, vals_hbm.at[wid])
    pltpu.sync_copy(sti_v, idx_hbm.at[wid])


@jax.jit
def kernel(logits, beam_scores):
    b, beam, vocab = logits.shape
    xT = jnp.transpose(logits, (1, 2, 0))

    sc_topk = functools.partial(
        pl.kernel,
        mesh=plsc.VectorSubcoreMesh(core_axis_name="c", subcore_axis_name="s"),
        compiler_params=pltpu.CompilerParams(
            needs_layout_passes=False, use_tc_tiling_on_sc=True),
        out_type=[
            jax.ShapeDtypeStruct((32, 4, b), jnp.float32),
            jax.ShapeDtypeStruct((32, 4, b), jnp.int32),
        ],
        scratch_types=[
            pltpu.VMEM((2, CHUNK, b), jnp.float32),
            pltpu.VMEM((NLG, 4, 16), jnp.int32),
            pltpu.VMEM((4, b), jnp.float32),
            pltpu.VMEM((4, b), jnp.int32),
            pltpu.SemaphoreType.DMA,
        ],
    )(_sc_topk_body)
    cvals, cidx = sc_topk(xT)

    mpart, spart = pl.pallas_call(
        _lse_part_kernel,
        grid=(beam, vocab // LSE_BLK),
        in_specs=[pl.BlockSpec((1, LSE_BLK, b), lambda i, j: (i, j, 0))],
        out_specs=[
            pl.BlockSpec((1, 1, 1, b), lambda i, j: (i, j, 0, 0)),
            pl.BlockSpec((1, 1, 1, b), lambda i, j: (i, j, 0, 0)),
        ],
        out_shape=[
            jax.ShapeDtypeStruct((beam, vocab // LSE_BLK, 1, b), jnp.float32),
            jax.ShapeDtypeStruct((beam, vocab // LSE_BLK, 1, b), jnp.float32),
        ],
    )(xT)

    cv = cvals.transpose(2, 0, 1).reshape(b, 128)
    ct = cidx.transpose(2, 0, 1).reshape(b, 128)
    mp = mpart.reshape(beam, vocab // LSE_BLK, b).transpose(2, 0, 1).reshape(
        b, beam * (vocab // LSE_BLK))
    sp = spart.reshape(beam, vocab // LSE_BLK, b).transpose(2, 0, 1).reshape(
        b, beam * (vocab // LSE_BLK))

    os_, ot, ob = pl.pallas_call(
        _merge_kernel,
        out_shape=[
            jax.ShapeDtypeStruct((b, 4), jnp.float32),
            jax.ShapeDtypeStruct((b, 4), jnp.int32),
            jax.ShapeDtypeStruct((b, 4), jnp.int32),
        ],
    )(cv, ct, mp, sp, beam_scores)

    return os_, ot, ob


# speedup vs baseline: 1.7829x; 1.0152x over previous
import functools

import jax
import jax.numpy as jnp
from jax import lax
from jax.experimental import pallas as pl
from jax.experimental.pallas import tpu as pltpu
from jax.experimental.pallas import tpu_sc as plsc

B = 128
BEAM = 4
VOCAB = 100000
NEG = -3.0e38
INTBIG = 2 ** 30

CHUNK = 400
NCH = VOCAB // CHUNK
NSTR = 8
NLG = 8
VUNROLL = 2

LSE_BLK = VOCAB // 20


def _lse_part_kernel(x_ref, m_ref, s_ref):
    x = x_ref[0]
    mx = jnp.max(x, axis=0, keepdims=True)
    s = jnp.sum(jnp.exp(x - mx), axis=0, keepdims=True)
    m_ref[...] = mx[None, None]
    s_ref[...] = s[None, None]


def _merge_kernel(cv_ref, ct_ref, mp_ref, sp_ref, bs_ref,
                  os_ref, ot_ref, ob_ref):
    cv = cv_ref[...]
    ct = ct_ref[...]
    mp = mp_ref[...]
    sp = sp_ref[...]
    bs = bs_ref[...]

    grp = jax.lax.broadcasted_iota(jnp.int32, mp.shape, 1) // 20
    slotbeam = jax.lax.broadcasted_iota(jnp.int32, cv.shape, 1) // 32

    adj = jnp.zeros_like(cv)
    for m in range(BEAM):
        sel = grp == m
        mb = jnp.max(jnp.where(sel, mp, NEG), axis=1, keepdims=True)
        sb = jnp.sum(jnp.where(sel, sp * jnp.exp(mp - mb), 0.0),
                     axis=1, keepdims=True)
        lse = mb + jnp.log(sb)
        adj = adj + jnp.where(slotbeam == m, bs[:, m:m + 1] - lse, 0.0)

    y = cv + adj
    fl = slotbeam * VOCAB + ct
    ss, tt, bb = [], [], []
    for _ in range(4):
        v = jnp.max(y, axis=1, keepdims=True)
        flb = jnp.min(jnp.where(y == v, fl, INTBIG), axis=1, keepdims=True)
        ss.append(v)
        tt.append(flb % VOCAB)
        bb.append(flb // VOCAB)
        y = jnp.where(fl == flb, NEG, y)
    os_ref[...] = jnp.concatenate(ss, axis=1)
    ot_ref[...] = jnp.concatenate(tt, axis=1)
    ob_ref[...] = jnp.concatenate(bb, axis=1)


def _sc_topk_body(x_hbm, vals_hbm, idx_hbm, buf_v, iv_v, stv_v, sti_v, sem):
    wid = lax.axis_index("s") * 2 + lax.axis_index("c")
    m = wid // NSTR
    j = wid - m * NSTR
    nk = (NCH - 1 - j) // NSTR + 1
    lane = lax.broadcasted_iota(jnp.int32, (16,), 0)
    negv = jnp.full((16,), NEG, jnp.float32)

    def chunk_src(k):
        return x_hbm.at[m, pl.ds(k * CHUNK, CHUNK)]

    pltpu.async_copy(chunk_src(j), buf_v.at[0], sem)

    def chunk_body(c, tq):
        k = j + NSTR * c
        sel = lax.rem(c, 2)
        knext = jnp.minimum(k + NSTR, NCH - 1)
        pltpu.async_copy(chunk_src(knext), buf_v.at[1 - sel], sem)
        pltpu.make_async_copy(chunk_src(k), buf_v.at[sel], sem).wait()

        def v_body(v, tq):
            vabs = k * CHUNK + v
            vs = [buf_v[sel, v, pl.ds(lg * 16, 16)] for lg in range(NLG)]
            tms = [vs[lg] > tq[lg * 4 + 3] for lg in range(NLG)]
            t01 = tms[0] | tms[1]
            t23 = tms[2] | tms[3]
            t45 = tms[4] | tms[5]
            t67 = tms[6] | tms[7]
            tor = (t01 | t23) | (t45 | t67)
            cnt = plsc.all_reduce_population_count(tor)[0]

            def slow(tq):
                out = list(tq)
                for lg in range(NLG):
                    clg = vs[lg] > tq[lg * 4 + 3]
                    cl = plsc.all_reduce_population_count(clg)[0]

                    def ins(q4, lg=lg):
                        q = list(q4)
                        t = vs[lg]
                        ti = lane * 0 + vabs
                        for r in range(4):
                            iv = iv_v[lg, r]
                            cc = t > q[r]
                            nv = jnp.where(cc, t, q[r])
                            ni = jnp.where(cc, ti, iv)
                            t = jnp.where(cc, q[r], t)
                            ti = jnp.where(cc, iv, ti)
                            q[r] = nv
                            iv_v[lg, r] = ni
                        return tuple(q)

                    q4 = lax.cond(cl > 0, ins, lambda q4: q4,
                                  tuple(out[lg * 4:lg * 4 + 4]))
                    out[lg * 4:lg * 4 + 4] = list(q4)
                return tuple(out)

            return lax.cond(cnt > 0, slow, lambda tq: tq, tq)

        return lax.fori_loop(0, CHUNK, v_body, tq)

    tq0 = (negv,) * (NLG * 4)
    tq = lax.fori_loop(0, nk, chunk_body, tq0)
    pltpu.make_async_copy(chunk_src(0), buf_v.at[0], sem).wait()

    for r in range(4):
        for lg in range(NLG):
            stv_v[r, pl.ds(lg * 16, 16)] = tq[lg * 4 + r]
            sti_v[r, pl.ds(lg * 16, 16)] = iv_v[lg, r]
    pltpu.sync_copy(stv_v, vals_hbm.at[wid])
    pltpu.sync_copy(sti_v, idx_hbm.at[wid])


@jax.jit
def kernel(logits, beam_scores):
    b, beam, vocab = logits.shape
    xT = jnp.transpose(logits, (1, 2, 0))

    sc_topk = functools.partial(
        pl.kernel,
        mesh=plsc.VectorSubcoreMesh(core_axis_name="c", subcore_axis_name="s"),
        compiler_params=pltpu.CompilerParams(
            needs_layout_passes=False, use_tc_tiling_on_sc=True),
        out_type=[
            jax.ShapeDtypeStruct((32, 4, b), jnp.float32),
            jax.ShapeDtypeStruct((32, 4, b), jnp.int32),
        ],
        scratch_types=[
            pltpu.VMEM((2, CHUNK, b), jnp.float32),
            pltpu.VMEM((NLG, 4, 16), jnp.int32),
            pltpu.VMEM((4, b), jnp.float32),
            pltpu.VMEM((4, b), jnp.int32),
            pltpu.SemaphoreType.DMA,
        ],
    )(_sc_topk_body)
    cvals, cidx = sc_topk(xT)

    mpart, spart = pl.pallas_call(
        _lse_part_kernel,
        grid=(beam, vocab // LSE_BLK),
        in_specs=[pl.BlockSpec((1, LSE_BLK, b), lambda i, j: (i, j, 0))],
        out_specs=[
            pl.BlockSpec((1, 1, 1, b), lambda i, j: (i, j, 0, 0)),
            pl.BlockSpec((1, 1, 1, b), lambda i, j: (i, j, 0, 0)),
        ],
        out_shape=[
            jax.ShapeDtypeStruct((beam, vocab // LSE_BLK, 1, b), jnp.float32),
            jax.ShapeDtypeStruct((beam, vocab // LSE_BLK, 1, b), jnp.float32),
        ],
    )(xT)

    cv = cvals.transpose(2, 0, 1).reshape(b, 128)
    ct = cidx.transpose(2, 0, 1).reshape(b, 128)
    mp = mpart.reshape(beam, vocab // LSE_BLK, b).transpose(2, 0, 1).reshape(
        b, beam * (vocab // LSE_BLK))
    sp = spart.reshape(beam, vocab // LSE_BLK, b).transpose(2, 0, 1).reshape(
        b, beam * (vocab // LSE_BLK))

    os_, ot, ob = pl.pallas_call(
        _merge_kernel,
        out_shape=[
            jax.ShapeDtypeStruct((b, 4), jnp.float32),
            jax.ShapeDtypeStruct((b, 4), jnp.int32),
            jax.ShapeDtypeStruct((b, 4), jnp.int32),
        ],
    )(cv, ct, mp, sp, beam_scores)

    return os_, ot, ob
